# baseline (speedup 1.0000x reference)
import functools

import jax
import jax.numpy as jnp
from jax.experimental import pallas as pl
from jax.experimental.pallas import tpu as pltpu

_B = 1024
_K = 400.0
_N = 500.0
_EPS = 0.1
_ITERS = 5

_NT = (((1,), (1,)), ((), ()))


def _soft_topk_mul(s):
    m = jnp.max(jnp.maximum(s, jnp.abs(s - 1.0)))
    a = 1.0 / (_EPS * m * m)
    q = jnp.exp((2.0 * s - 1.0) * a)

    def body(_, x):
        t = 1.0 / (q + x)
        s1 = jnp.sum(t, axis=1, keepdims=True)
        s2 = jnp.sum(t * t, axis=1, keepdims=True)
        xn = x - (x * s1 - (_N - _K)) / (s1 - x * s2)
        return jnp.abs(xn)

    x = jax.lax.fori_loop(0, _ITERS, body,
                          jnp.full((_B, 1), 1e-6, jnp.float32))
    t = 1.0 / (q + x)
    s1 = jnp.sum(t, axis=1, keepdims=True)
    s2 = jnp.sum(t * t, axis=1, keepdims=True)
    x = jnp.abs(x - (x * s1 - (_N - _K)) / (s1 - x * s2))
    return s - (s * x) * t


def _dot_nt(a, b):
    return jax.lax.dot_general(a, b, _NT, preferred_element_type=jnp.float32)


_KCH = 4
_KBLK = 1024 // _KCH


def _fwd(x_ref, w1_ref, b1_ref, w2_ref, b2_ref, w3_ref, b3_ref, w4_ref,
         b4_ref, o_ref, acc_ref):
    i = pl.program_id(0)
    part = _dot_nt(x_ref[...], w1_ref[...])

    @pl.when(i == 0)
    def _():
        acc_ref[...] = part

    @pl.when(i > 0)
    def _():
        acc_ref[...] += part

    @pl.when(i == _KCH - 1)
    def _():
        s = jnp.maximum(acc_ref[...] + b1_ref[...], 0.0)
        for w_ref, b_ref in ((w2_ref, b2_ref), (w3_ref, b3_ref)):
            h = _soft_topk_mul(s)
            s = jnp.maximum(_dot_nt(h, w_ref[...]) + b_ref[...], 0.0)
        h = _soft_topk_mul(s)
        o_ref[...] = _dot_nt(h, w4_ref[...]) + b4_ref[...]


@jax.jit
def kernel(x, W1, b1, W2, b2, W3, b3, W4, b4):
    full = lambda shape: pl.BlockSpec(shape, lambda i: (0, 0))
    nout = W4.shape[0]
    return pl.pallas_call(
        _fwd,
        grid=(_KCH,),
        in_specs=[
            pl.BlockSpec((_B, _KBLK), lambda i: (0, i)),
            pl.BlockSpec((int(_N), _KBLK), lambda i: (0, i)),
            full((1, int(_N))),
            full((int(_N), int(_N))), full((1, int(_N))),
            full((int(_N), int(_N))), full((1, int(_N))),
            full((nout, int(_N))), full((1, nout)),
        ],
        out_specs=full((_B, nout)),
        out_shape=jax.ShapeDtypeStruct((_B, nout), jnp.float32),
        scratch_shapes=[pltpu.VMEM((_B, int(_N)), jnp.float32)],
        compiler_params=pltpu.CompilerParams(
            dimension_semantics=("arbitrary",)),
    )(x, W1, b1.reshape(1, -1), W2, b2.reshape(1, -1), W3, b3.reshape(1, -1),
      W4, b4.reshape(1, -1))
